# Initial kernel scaffold; baseline (speedup 1.0000x reference)
import jax
import jax.numpy as jnp
from jax.experimental import pallas as pl


def kernel(x):
    raise NotImplementedError("write your pallas kernel here")


# speedup vs baseline: 20.9229x; 20.9229x over previous
import functools

import jax
import jax.numpy as jnp
from jax import lax
from jax.experimental import pallas as pl
from jax.experimental.pallas import tpu as pltpu
from jax.experimental.pallas import tpu_sc as plsc

N, C, H, W = 8, 96, 224, 224
ROW = H * W
ROWS = N * C
K = int(max(1, 0.1 * ROW))
LANES = 16
VPR = ROW // LANES
NBINS = 512
NCHUNK = NBINS // LANES
NWORKERS = 32
ROWS_PER = ROWS // NWORKERS
PAIRS = ROWS_PER // 2


def _tec_body(x_hbm, o_hbm, buf0, buf1, hist, si0, si1, so0, so1):
    wid = lax.axis_index("s") * 2 + lax.axis_index("c")
    base = wid * ROWS_PER
    lane = lax.iota(jnp.int32, LANES)
    ones = jnp.ones((LANES,), jnp.int32)
    zeros16 = jnp.zeros((LANES,), jnp.int32)

    def in_copy(buf, sem, r):
        return pltpu.make_async_copy(x_hbm.at[pl.ds(r * ROW, ROW)], buf, sem)

    def out_copy(buf, sem, r):
        return pltpu.make_async_copy(buf, o_hbm.at[pl.ds(r * ROW, ROW)], sem)

    def zero_hist():
        def zb(c, _):
            b = c * LANES
            for j in range(LANES):
                hist[b + j, :] = zeros16
            return 0
        lax.fori_loop(0, NCHUNK, zb, 0, unroll=2)

    def abs_bits(buf, i):
        v = buf[pl.ds(i * LANES, LANES)]
        return v, plsc.bitcast(v, jnp.int32) & jnp.int32(0x7FFFFFFF)

    def l1_pass(buf):
        def body(i, _):
            _, a = abs_bits(buf, i)
            plsc.addupdate_scatter(hist, [a >> 22, lane], ones)
            return 0
        lax.fori_loop(0, VPR, body, 0, unroll=8)

    def l23_pass(buf, prefix, shift):
        def body(i, _):
            _, a = abs_bits(buf, i)
            m = (a >> (shift + 9)) == prefix
            plsc.addupdate_scatter(hist, [(a >> shift) & (NBINS - 1), lane],
                                   ones, mask=m)
            return 0
        lax.fori_loop(0, VPR, body, 0, unroll=8)

    def search(kk):
        def cbody(cc, st):
            cnt, c_star, found = st
            c = (NCHUNK - 1) - cc
            b = c * LANES
            acc = hist[b, :]
            for j in range(1, LANES):
                acc = acc + hist[b + j, :]
            tot = jnp.sum(acc)
            hit = jnp.logical_and(jnp.logical_not(found), cnt + tot >= kk)
            c_star = jnp.where(hit, c, c_star)
            cnt = jnp.where(jnp.logical_or(found, hit), cnt, cnt + tot)
            return cnt, c_star, jnp.logical_or(found, hit)
        cnt, c_star, _ = lax.fori_loop(
            0, NCHUNK, cbody, (jnp.int32(0), jnp.int32(0), False))

        def bbody(bb, st):
            cnt, b_star, found = st
            b = (LANES - 1) - bb
            tot = jnp.sum(hist[c_star * LANES + b, :])
            hit = jnp.logical_and(jnp.logical_not(found), cnt + tot >= kk)
            b_star = jnp.where(hit, b, b_star)
            cnt = jnp.where(jnp.logical_or(found, hit), cnt, cnt + tot)
            return cnt, b_star, jnp.logical_or(found, hit)
        cnt2, b_star, _ = lax.fori_loop(
            0, LANES, bbody, (cnt, jnp.int32(0), False))
        return c_star * LANES + b_star, cnt2

    def find_threshold(buf):
        b1, above1 = search(K)
        zero_hist()
        l23_pass(buf, b1, 13)
        b2, above2 = search(K - above1)
        zero_hist()
        p2 = b1 * NBINS + b2
        l23_pass(buf, p2, 4)
        b3, _ = search(K - above1 - above2)
        zero_hist()
        return (p2 * NBINS + b3) * 16

    def mask_pass(buf, t):
        def body(i, _):
            v, a = abs_bits(buf, i)
            buf[pl.ds(i * LANES, LANES)] = jnp.where(a >= t, v, 0.0)
            return 0
        lax.fori_loop(0, VPR, body, 0, unroll=8)

    zero_hist()
    in_copy(buf0, si0, base).start()
    out_copy(buf1, so1, base + 1).start()

    def pair(j, _):
        r0 = base + 2 * j
        r1 = r0 + 1
        in_copy(buf0, si0, r0).wait()
        l1_pass(buf0)
        out_copy(buf1, so1, r1).wait()
        in_copy(buf1, si1, r1).start()
        t0 = find_threshold(buf0)
        mask_pass(buf0, t0)
        out_copy(buf0, so0, r0).start()
        in_copy(buf1, si1, r1).wait()
        l1_pass(buf1)
        out_copy(buf0, so0, r0).wait()
        rn = jnp.where(j < PAIRS - 1, r0 + 2, base)
        in_copy(buf0, si0, rn).start()
        t1 = find_threshold(buf1)
        mask_pass(buf1, t1)
        out_copy(buf1, so1, r1).start()
        return 0

    lax.fori_loop(0, PAIRS, pair, 0)
    in_copy(buf0, si0, base).wait()
    out_copy(buf1, so1, base + ROWS_PER - 1).wait()


@jax.jit
def kernel(x):
    xf = x.reshape(-1)
    run = pl.kernel(
        _tec_body,
        out_type=jax.ShapeDtypeStruct((ROWS * ROW,), jnp.float32),
        mesh=plsc.VectorSubcoreMesh(core_axis_name="c", subcore_axis_name="s"),
        scratch_types=[
            pltpu.VMEM((ROW,), jnp.float32),
            pltpu.VMEM((ROW,), jnp.float32),
            pltpu.VMEM((NBINS, LANES), jnp.int32),
            pltpu.SemaphoreType.DMA,
            pltpu.SemaphoreType.DMA,
            pltpu.SemaphoreType.DMA,
            pltpu.SemaphoreType.DMA,
        ],
        compiler_params=pltpu.CompilerParams(
            needs_layout_passes=False, use_tc_tiling_on_sc=False),
    )
    return run(xf).reshape(x.shape)


# speedup vs baseline: 62.6730x; 2.9954x over previous
import functools

import jax
import jax.numpy as jnp
from jax import lax
from jax.experimental import pallas as pl
from jax.experimental.pallas import tpu as pltpu
from jax.experimental.pallas import tpu_sc as plsc

N, C, H, W = 8, 96, 224, 224
ROW = H * W
ROWS = N * C
K = int(max(1, 0.1 * ROW))
LANES = 16
VPR = ROW // LANES
NBINS = 512
NCHUNK = NBINS // LANES
NWORKERS = 32
ROWS_PER = ROWS // NWORKERS
PAIRS = ROWS_PER // 2


def _tec_body(x_hbm, o_hbm, buf0, buf1, hist, si0, si1, so0, so1):
    wid = lax.axis_index("s") * 2 + lax.axis_index("c")
    base = wid * ROWS_PER
    lane = lax.iota(jnp.int32, LANES)
    ones = jnp.ones((LANES,), jnp.int32)
    zeros16 = jnp.zeros((LANES,), jnp.int32)

    def in_copy(buf, sem, r):
        return pltpu.make_async_copy(x_hbm.at[pl.ds(r * ROW, ROW)], buf, sem)

    def out_copy(buf, sem, r):
        return pltpu.make_async_copy(buf, o_hbm.at[pl.ds(r * ROW, ROW)], sem)

    def zero_hist():
        @plsc.parallel_loop(0, NBINS, step=LANES, unroll=2)
        def _(b):
            for j in range(LANES):
                hist[b + j, :] = zeros16

    def abs_bits(buf, i):
        v = buf[pl.ds(i * LANES, LANES)]
        return v, plsc.bitcast(v, jnp.int32) & jnp.int32(0x7FFFFFFF)

    def l1_pass(buf):
        @plsc.parallel_loop(0, VPR, unroll=8)
        def _(i):
            _, a = abs_bits(buf, i)
            plsc.addupdate_scatter(hist, [a >> 22, lane], ones)

    def l23_pass(buf, prefix, shift):
        @plsc.parallel_loop(0, VPR, unroll=8)
        def _(i):
            _, a = abs_bits(buf, i)
            m = (a >> (shift + 9)) == prefix
            plsc.addupdate_scatter(hist, [(a >> shift) & (NBINS - 1), lane],
                                   ones, mask=m)

    def search(kk):
        def cbody(cc, st):
            cnt, c_star, found = st
            c = (NCHUNK - 1) - cc
            b = c * LANES
            acc = hist[b, :]
            for j in range(1, LANES):
                acc = acc + hist[b + j, :]
            tot = jnp.sum(acc)
            hit = jnp.logical_and(jnp.logical_not(found), cnt + tot >= kk)
            c_star = jnp.where(hit, c, c_star)
            cnt = jnp.where(jnp.logical_or(found, hit), cnt, cnt + tot)
            return cnt, c_star, jnp.logical_or(found, hit)
        cnt, c_star, _ = lax.fori_loop(
            0, NCHUNK, cbody, (jnp.int32(0), jnp.int32(0), False))

        def bbody(bb, st):
            cnt, b_star, found = st
            b = (LANES - 1) - bb
            tot = jnp.sum(hist[c_star * LANES + b, :])
            hit = jnp.logical_and(jnp.logical_not(found), cnt + tot >= kk)
            b_star = jnp.where(hit, b, b_star)
            cnt = jnp.where(jnp.logical_or(found, hit), cnt, cnt + tot)
            return cnt, b_star, jnp.logical_or(found, hit)
        cnt2, b_star, _ = lax.fori_loop(
            0, LANES, bbody, (cnt, jnp.int32(0), False))
        return c_star * LANES + b_star, cnt2

    def find_threshold(buf):
        b1, above1 = search(K)
        zero_hist()
        l23_pass(buf, b1, 13)
        b2, above2 = search(K - above1)
        zero_hist()
        p2 = b1 * NBINS + b2
        l23_pass(buf, p2, 4)
        b3, _ = search(K - above1 - above2)
        zero_hist()
        return (p2 * NBINS + b3) * 16

    def mask_pass(buf, t):
        @plsc.parallel_loop(0, VPR, unroll=8)
        def _(i):
            v, a = abs_bits(buf, i)
            buf[pl.ds(i * LANES, LANES)] = jnp.where(a >= t, v, 0.0)

    zero_hist()
    in_copy(buf0, si0, base).start()
    out_copy(buf1, so1, base + 1).start()

    def pair(j, _):
        r0 = base + 2 * j
        r1 = r0 + 1
        in_copy(buf0, si0, r0).wait()
        l1_pass(buf0)
        out_copy(buf1, so1, r1).wait()
        in_copy(buf1, si1, r1).start()
        t0 = find_threshold(buf0)
        mask_pass(buf0, t0)
        out_copy(buf0, so0, r0).start()
        in_copy(buf1, si1, r1).wait()
        l1_pass(buf1)
        out_copy(buf0, so0, r0).wait()
        rn = jnp.where(j < PAIRS - 1, r0 + 2, base)
        in_copy(buf0, si0, rn).start()
        t1 = find_threshold(buf1)
        mask_pass(buf1, t1)
        out_copy(buf1, so1, r1).start()
        return 0

    lax.fori_loop(0, PAIRS, pair, 0)
    in_copy(buf0, si0, base).wait()
    out_copy(buf1, so1, base + ROWS_PER - 1).wait()


@jax.jit
def kernel(x):
    xf = x.reshape(-1)
    run = pl.kernel(
        _tec_body,
        out_type=jax.ShapeDtypeStruct((ROWS * ROW,), jnp.float32),
        mesh=plsc.VectorSubcoreMesh(core_axis_name="c", subcore_axis_name="s"),
        scratch_types=[
            pltpu.VMEM((ROW,), jnp.float32),
            pltpu.VMEM((ROW,), jnp.float32),
            pltpu.VMEM((NBINS, LANES), jnp.int32),
            pltpu.SemaphoreType.DMA,
            pltpu.SemaphoreType.DMA,
            pltpu.SemaphoreType.DMA,
            pltpu.SemaphoreType.DMA,
        ],
        compiler_params=pltpu.CompilerParams(
            needs_layout_passes=False, use_tc_tiling_on_sc=False),
    )
    return run(xf).reshape(x.shape)
